# baseline (device time: 9394 ns/iter reference)
import jax
import jax.numpy as jnp
from jax import lax
from jax.experimental import pallas as pl
from jax.experimental.pallas import tpu as pltpu

K = 8
IDX_BITS = 11
INT_MIN = jnp.iinfo(jnp.int32).min


def _mono(b):
    return b ^ ((b >> 31) & 0x7FFFFFFF)


def _extract_topk_cols(keys, k):
    cols = []
    cur = keys
    for _ in range(k):
        m = jnp.max(cur, axis=1, keepdims=True)
        cols.append(m)
        cur = jnp.where(cur == m, INT_MIN, cur)
    return cols


def _bitonic_merge_top8(a_desc, b_asc):
    m = jnp.maximum(a_desc, b_asc)
    lo, hi = m[:, :4], m[:, 4:]
    t = jnp.concatenate([jnp.maximum(lo, hi), jnp.minimum(lo, hi)], axis=1)
    parts = []
    for g in range(2):
        h = t[:, g * 4:(g + 1) * 4]
        parts += [jnp.maximum(h[:, :2], h[:, 2:]), jnp.minimum(h[:, :2], h[:, 2:])]
    t = jnp.concatenate(parts, axis=1)
    parts = []
    for g in range(4):
        a, b = t[:, 2 * g:2 * g + 1], t[:, 2 * g + 1:2 * g + 2]
        parts += [jnp.maximum(a, b), jnp.minimum(a, b)]
    return jnp.concatenate(parts, axis=1)


def kernel(x):
    m, n = x.shape

    def body(x_ref, out_ref, comm_ref, send_sem, recv_sem):
        my_x = lax.axis_index("x")
        my_y = lax.axis_index("y")
        nbr = (my_x, 1 - my_y)

        barrier_sem = pltpu.get_barrier_semaphore()
        pl.semaphore_signal(
            barrier_sem, inc=1, device_id=nbr,
            device_id_type=pl.DeviceIdType.MESH,
        )

        bits = _mono(lax.bitcast_convert_type(x_ref[:, :], jnp.int32))
        iota = lax.broadcasted_iota(jnp.int32, (m, n), 1)
        inv_base = (2 * n - 1) - my_y * n
        keys = ((bits >> IDX_BITS) << IDX_BITS) | (inv_base - iota)

        cols = _extract_topk_cols(keys, K)
        mine_desc = jnp.concatenate(cols, axis=1)
        comm_ref[0, :, :] = jnp.concatenate(cols[::-1], axis=1)

        pl.semaphore_wait(barrier_sem, 1)

        rdma = pltpu.make_async_remote_copy(
            src_ref=comm_ref.at[0],
            dst_ref=comm_ref.at[1],
            send_sem=send_sem,
            recv_sem=recv_sem,
            device_id=nbr,
            device_id_type=pl.DeviceIdType.MESH,
        )
        rdma.start()
        rdma.wait()

        top = _bitonic_merge_top8(mine_desc, comm_ref[1, :, :])
        vbits = _mono((top >> IDX_BITS) << IDX_BITS)
        out_ref[:, :] = lax.bitcast_convert_type(vbits, jnp.float32)

    return pl.pallas_call(
        body,
        out_shape=jax.ShapeDtypeStruct((m, K), jnp.float32),
        in_specs=[pl.BlockSpec(memory_space=pltpu.VMEM)],
        out_specs=pl.BlockSpec(memory_space=pltpu.VMEM),
        scratch_shapes=[
            pltpu.VMEM((2, m, K), jnp.int32),
            pltpu.SemaphoreType.DMA,
            pltpu.SemaphoreType.DMA,
        ],
        compiler_params=pltpu.CompilerParams(collective_id=0),
    )(x)


# device time: 9299 ns/iter; 1.0102x vs baseline; 1.0102x over previous
import jax
import jax.numpy as jnp
from jax import lax
from jax.experimental import pallas as pl
from jax.experimental.pallas import tpu as pltpu

K = 8
IDX_BITS = 11
KEY_MASK = -1 << IDX_BITS
INT_MIN = jnp.iinfo(jnp.int32).min


def _mono(b):
    return b ^ ((b >> 31) & 0x7FFFFFFF)


def _extract_topk_cols(keys, k):
    cols = []
    cur = keys
    for _ in range(k):
        m = jnp.max(cur, axis=1, keepdims=True)
        cols.append(m)
        cur = jnp.where(cur == m, INT_MIN, cur)
    return cols


def _bitonic_merge_top8(a_desc, b_asc):
    m = jnp.maximum(a_desc, b_asc)
    lo, hi = m[:, :4], m[:, 4:]
    t = jnp.concatenate([jnp.maximum(lo, hi), jnp.minimum(lo, hi)], axis=1)
    parts = []
    for g in range(2):
        h = t[:, g * 4:(g + 1) * 4]
        parts += [jnp.maximum(h[:, :2], h[:, 2:]), jnp.minimum(h[:, :2], h[:, 2:])]
    t = jnp.concatenate(parts, axis=1)
    parts = []
    for g in range(4):
        a, b = t[:, 2 * g:2 * g + 1], t[:, 2 * g + 1:2 * g + 2]
        parts += [jnp.maximum(a, b), jnp.minimum(a, b)]
    return jnp.concatenate(parts, axis=1)


def kernel(x):
    m, n = x.shape

    def body(x_ref, out_ref, comm_ref, send_sem, recv_sem):
        my_x = lax.axis_index("x")
        my_y = lax.axis_index("y")
        nbr = (my_x, 1 - my_y)

        barrier_sem = pltpu.get_barrier_semaphore()
        pl.semaphore_signal(
            barrier_sem, inc=1, device_id=nbr,
            device_id_type=pl.DeviceIdType.MESH,
        )

        bits = _mono(lax.bitcast_convert_type(x_ref[:, :], jnp.int32))
        keys = bits & KEY_MASK

        cols = _extract_topk_cols(keys, K)
        mine_desc = jnp.concatenate(cols, axis=1)
        comm_ref[0, :, :] = jnp.concatenate(cols[::-1], axis=1)

        pl.semaphore_wait(barrier_sem, 1)

        rdma = pltpu.make_async_remote_copy(
            src_ref=comm_ref.at[0],
            dst_ref=comm_ref.at[1],
            send_sem=send_sem,
            recv_sem=recv_sem,
            device_id=nbr,
            device_id_type=pl.DeviceIdType.MESH,
        )
        rdma.start()
        rdma.wait()

        top = _bitonic_merge_top8(mine_desc, comm_ref[1, :, :])
        out_ref[:, :] = lax.bitcast_convert_type(_mono(top), jnp.float32)

    return pl.pallas_call(
        body,
        out_shape=jax.ShapeDtypeStruct((m, K), jnp.float32),
        in_specs=[pl.BlockSpec(memory_space=pltpu.VMEM)],
        out_specs=pl.BlockSpec(memory_space=pltpu.VMEM),
        scratch_shapes=[
            pltpu.VMEM((2, m, K), jnp.int32),
            pltpu.SemaphoreType.DMA,
            pltpu.SemaphoreType.DMA,
        ],
        compiler_params=pltpu.CompilerParams(collective_id=0),
    )(x)
